# Initial kernel scaffold; baseline (speedup 1.0000x reference)
import jax
import jax.numpy as jnp
from jax.experimental import pallas as pl


def kernel(x_u, x_v, x_y, edge_index_u, edge_index_v, edge_index_y, nexus, W1, b1, W2, b2, We, be, Wn1, bn1, Wn2, bn2):
    raise NotImplementedError("write your pallas kernel here")


# speedup vs baseline: 34.5576x; 34.5576x over previous
import functools
import numpy as np
import jax
import jax.numpy as jnp
from jax import lax
from jax.experimental import pallas as pl
from jax.experimental.pallas import tpu as pltpu
from jax.experimental.pallas import tpu_sc as plsc

N = 10000
C = 5
PF = 64
NF = 16
D1 = C * NF
DT = 96
DX = C * PF
NC = 2
NS = 16
NW = NC * NS
SW = 128
N2 = 10240
RPT = N2 // NS
BN = 512
GRID = N2 // BN

f32 = jnp.float32
i32 = jnp.int32


def _blkdiag(w):
    return jax.scipy.linalg.block_diag(*[w[c] for c in range(w.shape[0])])


def _tc1_body(xu, xv, xy, w1b, wa, bea, yu, yv, yy, au, av, ay):
    for p, (x, yo, ao) in enumerate(((xu, yu, au), (xv, yv, av), (xy, yy, ay))):
        xb = x[...]
        yo[...] = jnp.dot(xb, w1b[p], preferred_element_type=f32)
        ao[...] = jnp.dot(xb, wa[p], preferred_element_type=f32) + bea[p]


def _tc1(xs, w1b, wa, bea):
    full = lambda s: pl.BlockSpec(s, lambda i: (0,) * len(s))
    return pl.pallas_call(
        _tc1_body,
        grid=(GRID,),
        in_specs=[pl.BlockSpec((BN, DX), lambda i: (i, 0))] * 3 + [
            full((3, DX, D1)), full((3, DX, 16)), full((3, 16))],
        out_specs=[pl.BlockSpec((BN, D1), lambda i: (i, 0))] * 3 +
                  [pl.BlockSpec((BN, 16), lambda i: (i, 0))] * 3,
        out_shape=[jax.ShapeDtypeStruct((N2, D1), f32)] * 3 +
                  [jax.ShapeDtypeStruct((N2, 16), f32)] * 3,
    )(*xs, w1b, wa, bea)


_MESH = plsc.VectorSubcoreMesh(core_axis_name="c", subcore_axis_name="s")


@functools.partial(
    pl.kernel,
    out_type=jax.ShapeDtypeStruct((NC, N2, D1), f32),
    mesh=_MESH,
    compiler_params=pltpu.CompilerParams(use_tc_tiling_on_sc=False, needs_layout_passes=False),
    scratch_types=[
        pltpu.VMEM((40, SW), i32),
        pltpu.VMEM((40, SW), i32),
        pltpu.VMEM((SW, D1), f32),
        pltpu.VMEM_SHARED((N2, D1), f32),
        pltpu.SemaphoreType.DMA,
    ],
)
def _sc_phase1(yu, yv, yy, su, du, sv, dv, sy, dy, z80, out,
               idx_s, idx_d, buf, acc, sem):
    cid = lax.axis_index("c")
    sid = lax.axis_index("s")
    wid = sid * NC + cid
    pltpu.sync_copy(z80.at[pl.ds(sid * RPT, RPT)], acc.at[pl.ds(sid * RPT, RPT)])
    plsc.subcore_barrier()
    for (y, s2d, d2d) in ((yu, su, du), (yv, sv, dv), (yy, sy, dy)):
        pltpu.sync_copy(s2d.at[pl.ds(wid * 40, 40)], idx_s)
        pltpu.sync_copy(d2d.at[pl.ds(wid * 40, 40)], idx_d)

        def chunk(j, carry):
            pltpu.async_copy(y.at[idx_s.at[j]], buf, sem).wait()
            pltpu.sync_copy(buf, acc.at[idx_d.at[j]], add=True)
            return carry

        lax.fori_loop(0, 40, chunk, 0)
    plsc.subcore_barrier()
    pltpu.sync_copy(acc.at[pl.ds(sid * RPT, RPT)],
                    out.at[cid, pl.ds(sid * RPT, RPT)])


def _tc2_body(part, b1r, b2r, w2b, wb, tu, tv, ty):
    n1 = jnp.tanh(part[0] + part[1] + b1r[...])
    n2 = jnp.tanh(jnp.dot(n1, w2b[...], preferred_element_type=f32) + b2r[...])
    for p, to in enumerate((tu, tv, ty)):
        bp = jnp.dot(n2, wb[p], preferred_element_type=f32)
        to[...] = jnp.concatenate([n2, bp], axis=1)


def _tc2(part1, b1r, b2r, w2b, wb):
    full = lambda s: pl.BlockSpec(s, lambda i: (0,) * len(s))
    return pl.pallas_call(
        _tc2_body,
        grid=(GRID,),
        in_specs=[pl.BlockSpec((NC, BN, D1), lambda i: (0, i, 0)),
                  full((1, D1)), full((1, D1)), full((D1, D1)),
                  full((3, D1, 16))],
        out_specs=[pl.BlockSpec((BN, DT), lambda i: (i, 0))] * 3,
        out_shape=[jax.ShapeDtypeStruct((N2, DT), f32)] * 3,
    )(part1, b1r, b2r, w2b, wb)


@functools.partial(
    pl.kernel,
    out_type=jax.ShapeDtypeStruct((3, NC, N2, DT), f32),
    mesh=_MESH,
    compiler_params=pltpu.CompilerParams(use_tc_tiling_on_sc=False, needs_layout_passes=False),
    scratch_types=[
        pltpu.VMEM((40, SW), i32),
        pltpu.VMEM((40, SW), i32),
        pltpu.VMEM((SW, DT), f32),
        pltpu.VMEM((SW, 16), f32),
        pltpu.VMEM_SHARED((N2, DT), f32),
        pltpu.SemaphoreType.DMA,
        pltpu.SemaphoreType.DMA,
    ],
)
def _sc_phase2(tu, tv, ty, au, av, ay, su, du, sv, dv, sy, dy, z96, out,
               idx_s, idx_d, nbuf, abuf, acc, sem1, sem2):
    cid = lax.axis_index("c")
    sid = lax.axis_index("s")
    wid = sid * NC + cid
    lanes = lax.iota(i32, 16)
    ones = jnp.full((16,), 1.0, f32)
    for p, (tab, atab, s2d, d2d) in enumerate(
            ((tu, au, su, du), (tv, av, sv, dv), (ty, ay, sy, dy))):
        pltpu.sync_copy(z96.at[pl.ds(sid * RPT, RPT)],
                        acc.at[pl.ds(sid * RPT, RPT)])
        plsc.subcore_barrier()
        pltpu.sync_copy(s2d.at[pl.ds(wid * 40, 40)], idx_s)
        pltpu.sync_copy(d2d.at[pl.ds(wid * 40, 40)], idx_d)

        def chunk(j, carry):
            ga = pltpu.async_copy(atab.at[idx_s.at[j]], abuf, sem1)
            gn = pltpu.async_copy(tab.at[idx_d.at[j]], nbuf, sem2)
            ga.wait()
            gn.wait()

            def group(g, c2):
                rows = g * 16 + lanes
                logits = []
                for c in range(C):
                    a = plsc.load_gather(abuf, [rows, jnp.full((16,), c, i32)])
                    b = plsc.load_gather(nbuf, [rows, jnp.full((16,), D1 + c, i32)])
                    logits.append(a + b)
                m = logits[0]
                for c in range(1, C):
                    m = jnp.maximum(m, logits[c])
                es = [jnp.exp(l - m) for l in logits]
                tot = es[0]
                for c in range(1, C):
                    tot = tot + es[c]
                inv = 1.0 / tot
                plsc.store_scatter(nbuf, [rows, jnp.full((16,), 85, i32)], ones)
                for c in range(C):
                    ec = es[c] * inv
                    for f in range(NF):
                        col = jnp.full((16,), c * NF + f, i32)
                        v = plsc.load_gather(nbuf, [rows, col])
                        plsc.store_scatter(nbuf, [rows, col], v * ec)
                return c2

            lax.fori_loop(0, SW // 16, group, 0)
            pltpu.sync_copy(nbuf, acc.at[idx_s.at[j]], add=True)
            return carry

        lax.fori_loop(0, 40, chunk, 0)
        plsc.subcore_barrier()
        pltpu.sync_copy(acc.at[pl.ds(sid * RPT, RPT)],
                        out.at[p, cid, pl.ds(sid * RPT, RPT)])
        plsc.subcore_barrier()


def _tc3_body(xu, xv, xy, part, wn1b, bn1r, wn2b, bn2r, ou, ov, oy):
    for p, (x, o) in enumerate(((xu, ou), (xv, ov), (xy, oy))):
        s = part[p, 0, :, :D1] + part[p, 1, :, :D1]
        cnt = part[p, 0, :, 85:86] + part[p, 1, :, 85:86]
        aggr = s / jnp.maximum(cnt, 1.0)
        h = jnp.concatenate([x[...], aggr], axis=1)
        h = jnp.tanh(jnp.dot(h, wn1b[p], preferred_element_type=f32) + bn1r[p])
        h = jnp.tanh(jnp.dot(h, wn2b[p], preferred_element_type=f32) + bn2r[p])
        o[...] = h


def _tc3(xs, part2, wn1b, bn1r, wn2b, bn2r):
    full = lambda s: pl.BlockSpec(s, lambda i: (0,) * len(s))
    return pl.pallas_call(
        _tc3_body,
        grid=(GRID,),
        in_specs=[pl.BlockSpec((BN, DX), lambda i: (i, 0))] * 3 + [
            pl.BlockSpec((3, NC, BN, DT), lambda i: (0, 0, i, 0)),
            full((3, DX + D1, DX)), full((3, 1, DX)),
            full((3, DX, DX)), full((3, 1, DX))],
        out_specs=[pl.BlockSpec((BN, DX), lambda i: (i, 0))] * 3,
        out_shape=[jax.ShapeDtypeStruct((N2, DX), f32)] * 3,
    )(*xs, part2, wn1b, bn1r, wn2b, bn2r)


def kernel(x_u, x_v, x_y, edge_index_u, edge_index_v, edge_index_y, nexus,
           W1, b1, W2, b2, We, be, Wn1, bn1, Wn2, bn2):
    E = edge_index_u.shape[1]
    E2 = ((E + NW * SW - 1) // (NW * SW)) * (NW * SW)

    xs = [jnp.pad(x.reshape(N, DX), ((0, N2 - N), (0, 0)))
          for x in (x_u, x_v, x_y)]
    eidx = []
    for ei in (edge_index_u, edge_index_v, edge_index_y):
        s = jnp.pad(ei[0], (0, E2 - E), constant_values=N).reshape(-1, SW)
        d = jnp.pad(ei[1], (0, E2 - E), constant_values=N).reshape(-1, SW)
        eidx += [s, d]
    z80 = jnp.zeros((N2, D1), f32)
    z96 = jnp.zeros((N2, DT), f32)

    w1b = jnp.stack([_blkdiag(W1[:, p * PF:(p + 1) * PF, :]) for p in range(3)])
    wa = jnp.stack([
        jnp.pad(_blkdiag(We[p, :, :PF, :]), ((0, 0), (0, 16 - C)))
        for p in range(3)])
    bea = jnp.pad(be[:, :, 0], ((0, 0), (0, 16 - C)))
    b1r = b1.reshape(1, D1)
    b2r = b2.reshape(1, D1)
    w2b = _blkdiag(W2)
    wb = jnp.stack([
        jnp.pad(_blkdiag(We[p, :, PF:, :]), ((0, 0), (0, 16 - C)))
        for p in range(3)])
    perm = np.empty((DX + D1,), np.int32)
    for c in range(C):
        perm[c * PF:(c + 1) * PF] = c * (PF + NF) + np.arange(PF)
        perm[DX + c * NF:DX + (c + 1) * NF] = c * (PF + NF) + PF + np.arange(NF)
    wn1b = jnp.stack([jnp.take(_blkdiag(Wn1[p]), perm, axis=0)
                      for p in range(3)])
    bn1r = bn1.reshape(3, 1, DX)
    wn2b = jnp.stack([_blkdiag(Wn2[p]) for p in range(3)])
    bn2r = bn2.reshape(3, 1, DX)

    yu, yv, yy, au, av, ay = _tc1(xs, w1b, wa, bea)
    part1 = _sc_phase1(yu, yv, yy, *eidx, z80)
    ntabs = _tc2(part1, b1r, b2r, w2b, wb)
    part2 = _sc_phase2(*ntabs, au, av, ay, *eidx, z96)
    outs = _tc3(xs, part2, wn1b, bn1r, wn2b, bn2r)
    return tuple(o[:N].reshape(N, C, PF) for o in outs)


# speedup vs baseline: 49.3361x; 1.4276x over previous
import functools
import numpy as np
import jax
import jax.numpy as jnp
from jax import lax
from jax.experimental import pallas as pl
from jax.experimental.pallas import tpu as pltpu
from jax.experimental.pallas import tpu_sc as plsc

N = 10000
C = 5
PF = 64
NF = 16
D1 = C * NF
DT = 96
DX = C * PF
NC = 2
NS = 16
NW = NC * NS
SW = 128
N2 = 10240
RPT = N2 // NS
BN = 512
GRID = N2 // BN

f32 = jnp.float32
i32 = jnp.int32


def _blkdiag(w):
    return jax.scipy.linalg.block_diag(*[w[c] for c in range(w.shape[0])])


def _tc1_body(xu, xv, xy, w1b, wa, bea, yu, yv, yy, au, av, ay):
    for p, (x, yo, ao) in enumerate(((xu, yu, au), (xv, yv, av), (xy, yy, ay))):
        xb = x[...]
        yo[...] = jnp.dot(xb, w1b[p], preferred_element_type=f32)
        ao[...] = jnp.dot(xb, wa[p], preferred_element_type=f32) + bea[p]


def _tc1(xs, w1b, wa, bea):
    full = lambda s: pl.BlockSpec(s, lambda i: (0,) * len(s))
    return pl.pallas_call(
        _tc1_body,
        grid=(GRID,),
        in_specs=[pl.BlockSpec((BN, DX), lambda i: (i, 0))] * 3 + [
            full((3, DX, D1)), full((3, DX, 16)), full((3, 16))],
        out_specs=[pl.BlockSpec((BN, D1), lambda i: (i, 0))] * 3 +
                  [pl.BlockSpec((BN, 16), lambda i: (i, 0))] * 3,
        out_shape=[jax.ShapeDtypeStruct((N2, D1), f32)] * 3 +
                  [jax.ShapeDtypeStruct((N2, 16), f32)] * 3,
    )(*xs, w1b, wa, bea)


_MESH = plsc.VectorSubcoreMesh(core_axis_name="c", subcore_axis_name="s")


@functools.partial(
    pl.kernel,
    out_type=jax.ShapeDtypeStruct((NC, N2, D1), f32),
    mesh=_MESH,
    compiler_params=pltpu.CompilerParams(use_tc_tiling_on_sc=False, needs_layout_passes=False),
    scratch_types=[
        pltpu.VMEM((40, SW), i32),
        pltpu.VMEM((40, SW), i32),
        pltpu.VMEM((SW, D1), f32),
        pltpu.VMEM_SHARED((N2, D1), f32),
        pltpu.SemaphoreType.DMA,
    ],
)
def _sc_phase1(yu, yv, yy, su, du, sv, dv, sy, dy, z80, out,
               idx_s, idx_d, buf, acc, sem):
    cid = lax.axis_index("c")
    sid = lax.axis_index("s")
    wid = sid * NC + cid
    pltpu.sync_copy(z80.at[pl.ds(sid * RPT, RPT)], acc.at[pl.ds(sid * RPT, RPT)])
    plsc.subcore_barrier()
    for (y, s2d, d2d) in ((yu, su, du), (yv, sv, dv), (yy, sy, dy)):
        pltpu.sync_copy(s2d.at[pl.ds(wid * 40, 40)], idx_s)
        pltpu.sync_copy(d2d.at[pl.ds(wid * 40, 40)], idx_d)

        def chunk(j, carry):
            pltpu.async_copy(y.at[idx_s.at[j]], buf, sem).wait()
            pltpu.sync_copy(buf, acc.at[idx_d.at[j]], add=True)
            return carry

        lax.fori_loop(0, 40, chunk, 0)
    plsc.subcore_barrier()
    pltpu.sync_copy(acc.at[pl.ds(sid * RPT, RPT)],
                    out.at[cid, pl.ds(sid * RPT, RPT)])


def _tc2_body(part, b1r, b2r, w2b, wb, tu, tv, ty):
    n1 = jnp.tanh(part[0] + part[1] + b1r[...])
    n2 = jnp.tanh(jnp.dot(n1, w2b[...], preferred_element_type=f32) + b2r[...])
    for p, to in enumerate((tu, tv, ty)):
        bp = jnp.dot(n2, wb[p], preferred_element_type=f32)
        to[...] = jnp.concatenate([n2, bp], axis=1)


def _tc2(part1, b1r, b2r, w2b, wb):
    full = lambda s: pl.BlockSpec(s, lambda i: (0,) * len(s))
    return pl.pallas_call(
        _tc2_body,
        grid=(GRID,),
        in_specs=[pl.BlockSpec((NC, BN, D1), lambda i: (0, i, 0)),
                  full((1, D1)), full((1, D1)), full((D1, D1)),
                  full((3, D1, 16))],
        out_specs=[pl.BlockSpec((BN, DT), lambda i: (i, 0))] * 3,
        out_shape=[jax.ShapeDtypeStruct((N2, DT), f32)] * 3,
    )(part1, b1r, b2r, w2b, wb)


@functools.partial(
    pl.kernel,
    out_type=jax.ShapeDtypeStruct((3, NC, N2, DT), f32),
    mesh=_MESH,
    compiler_params=pltpu.CompilerParams(use_tc_tiling_on_sc=False, needs_layout_passes=False),
    scratch_types=[
        pltpu.VMEM((40, SW), i32),
        pltpu.VMEM((40, SW), i32),
        pltpu.VMEM((SW, DT), f32),
        pltpu.VMEM((SW, 16), f32),
        pltpu.VMEM_SHARED((N2, DT), f32),
        pltpu.SemaphoreType.DMA,
        pltpu.SemaphoreType.DMA,
    ],
)
def _sc_phase2(tu, tv, ty, au, av, ay, su, du, sv, dv, sy, dy, z96, out,
               idx_s, idx_d, nbuf, abuf, acc, sem1, sem2):
    cid = lax.axis_index("c")
    sid = lax.axis_index("s")
    wid = sid * NC + cid
    lane_lt5 = lax.iota(i32, 16) < C
    cntvec = jnp.where(lax.iota(i32, 16) == (85 - D1), 1.0, 0.0).astype(f32)
    for p, (tab, atab, s2d, d2d) in enumerate(
            ((tu, au, su, du), (tv, av, sv, dv), (ty, ay, sy, dy))):
        pltpu.sync_copy(z96.at[pl.ds(sid * RPT, RPT)],
                        acc.at[pl.ds(sid * RPT, RPT)])
        plsc.subcore_barrier()
        pltpu.sync_copy(s2d.at[pl.ds(wid * 40, 40)], idx_s)
        pltpu.sync_copy(d2d.at[pl.ds(wid * 40, 40)], idx_d)

        def chunk(j, carry):
            ga = pltpu.async_copy(atab.at[idx_s.at[j]], abuf, sem1)
            gn = pltpu.async_copy(tab.at[idx_d.at[j]], nbuf, sem2)
            ga.wait()
            gn.wait()

            def edge(e, c2):
                a = abuf[e, :]
                b = nbuf[e, pl.ds(D1, 16)]
                l = jnp.where(lane_lt5, a + b, -1e30)
                ex = jnp.exp(l - jnp.max(l))
                sc = ex / jnp.sum(ex)
                for c in range(C):
                    v = nbuf[e, pl.ds(c * NF, NF)]
                    nbuf[e, pl.ds(c * NF, NF)] = v * sc[c]
                nbuf[e, pl.ds(D1, 16)] = cntvec
                return c2

            lax.fori_loop(0, SW, edge, 0)
            pltpu.sync_copy(nbuf, acc.at[idx_s.at[j]], add=True)
            return carry

        lax.fori_loop(0, 40, chunk, 0)
        plsc.subcore_barrier()
        pltpu.sync_copy(acc.at[pl.ds(sid * RPT, RPT)],
                        out.at[p, cid, pl.ds(sid * RPT, RPT)])
        plsc.subcore_barrier()


def _tc3_body(xu, xv, xy, part, wn1b, bn1r, wn2b, bn2r, ou, ov, oy):
    for p, (x, o) in enumerate(((xu, ou), (xv, ov), (xy, oy))):
        s = part[p, 0, :, :D1] + part[p, 1, :, :D1]
        cnt = part[p, 0, :, 85:86] + part[p, 1, :, 85:86]
        aggr = s / jnp.maximum(cnt, 1.0)
        h = jnp.concatenate([x[...], aggr], axis=1)
        h = jnp.tanh(jnp.dot(h, wn1b[p], preferred_element_type=f32) + bn1r[p])
        h = jnp.tanh(jnp.dot(h, wn2b[p], preferred_element_type=f32) + bn2r[p])
        o[...] = h


def _tc3(xs, part2, wn1b, bn1r, wn2b, bn2r):
    full = lambda s: pl.BlockSpec(s, lambda i: (0,) * len(s))
    return pl.pallas_call(
        _tc3_body,
        grid=(GRID,),
        in_specs=[pl.BlockSpec((BN, DX), lambda i: (i, 0))] * 3 + [
            pl.BlockSpec((3, NC, BN, DT), lambda i: (0, 0, i, 0)),
            full((3, DX + D1, DX)), full((3, 1, DX)),
            full((3, DX, DX)), full((3, 1, DX))],
        out_specs=[pl.BlockSpec((BN, DX), lambda i: (i, 0))] * 3,
        out_shape=[jax.ShapeDtypeStruct((N2, DX), f32)] * 3,
    )(*xs, part2, wn1b, bn1r, wn2b, bn2r)


def kernel(x_u, x_v, x_y, edge_index_u, edge_index_v, edge_index_y, nexus,
           W1, b1, W2, b2, We, be, Wn1, bn1, Wn2, bn2):
    E = edge_index_u.shape[1]
    E2 = ((E + NW * SW - 1) // (NW * SW)) * (NW * SW)

    xs = [jnp.pad(x.reshape(N, DX), ((0, N2 - N), (0, 0)))
          for x in (x_u, x_v, x_y)]
    eidx = []
    for ei in (edge_index_u, edge_index_v, edge_index_y):
        s = jnp.pad(ei[0], (0, E2 - E), constant_values=N).reshape(-1, SW)
        d = jnp.pad(ei[1], (0, E2 - E), constant_values=N).reshape(-1, SW)
        eidx += [s, d]
    z80 = jnp.zeros((N2, D1), f32)
    z96 = jnp.zeros((N2, DT), f32)

    w1b = jnp.stack([_blkdiag(W1[:, p * PF:(p + 1) * PF, :]) for p in range(3)])
    wa = jnp.stack([
        jnp.pad(_blkdiag(We[p, :, :PF, :]), ((0, 0), (0, 16 - C)))
        for p in range(3)])
    bea = jnp.pad(be[:, :, 0], ((0, 0), (0, 16 - C)))
    b1r = b1.reshape(1, D1)
    b2r = b2.reshape(1, D1)
    w2b = _blkdiag(W2)
    wb = jnp.stack([
        jnp.pad(_blkdiag(We[p, :, PF:, :]), ((0, 0), (0, 16 - C)))
        for p in range(3)])
    perm = np.empty((DX + D1,), np.int32)
    for c in range(C):
        perm[c * PF:(c + 1) * PF] = c * (PF + NF) + np.arange(PF)
        perm[DX + c * NF:DX + (c + 1) * NF] = c * (PF + NF) + PF + np.arange(NF)
    wn1b = jnp.stack([jnp.take(_blkdiag(Wn1[p]), perm, axis=0)
                      for p in range(3)])
    bn1r = bn1.reshape(3, 1, DX)
    wn2b = jnp.stack([_blkdiag(Wn2[p]) for p in range(3)])
    bn2r = bn2.reshape(3, 1, DX)

    yu, yv, yy, au, av, ay = _tc1(xs, w1b, wa, bea)
    part1 = _sc_phase1(yu, yv, yy, *eidx, z80)
    ntabs = _tc2(part1, b1r, b2r, w2b, wb)
    part2 = _sc_phase2(*ntabs, au, av, ay, *eidx, z96)
    outs = _tc3(xs, part2, wn1b, bn1r, wn2b, bn2r)
    return tuple(o[:N].reshape(N, C, PF) for o in outs)


# speedup vs baseline: 56.9062x; 1.1534x over previous
import functools
import numpy as np
import jax
import jax.numpy as jnp
from jax import lax
from jax.experimental import pallas as pl
from jax.experimental.pallas import tpu as pltpu
from jax.experimental.pallas import tpu_sc as plsc

N = 10000
C = 5
PF = 64
NF = 16
D1 = C * NF
DT = 96
DX = C * PF
NC = 2
NS = 16
NW = NC * NS
SW = 128
N2 = 10240
RPT = N2 // NS
BN = 512
GRID = N2 // BN

f32 = jnp.float32
i32 = jnp.int32


def _blkdiag(w):
    return jax.scipy.linalg.block_diag(*[w[c] for c in range(w.shape[0])])


def _tc1_body(xu, xv, xy, w1b, wa, bea, yu, yv, yy, au, av, ay):
    for p, (x, yo, ao) in enumerate(((xu, yu, au), (xv, yv, av), (xy, yy, ay))):
        xb = x[...]
        yo[...] = jnp.dot(xb, w1b[p], preferred_element_type=f32)
        ao[...] = jnp.dot(xb, wa[p], preferred_element_type=f32) + bea[p]


def _tc1(xs, w1b, wa, bea):
    full = lambda s: pl.BlockSpec(s, lambda i: (0,) * len(s))
    return pl.pallas_call(
        _tc1_body,
        grid=(GRID,),
        in_specs=[pl.BlockSpec((BN, DX), lambda i: (i, 0))] * 3 + [
            full((3, DX, D1)), full((3, DX, 16)), full((3, 16))],
        out_specs=[pl.BlockSpec((BN, D1), lambda i: (i, 0))] * 3 +
                  [pl.BlockSpec((BN, 16), lambda i: (i, 0))] * 3,
        out_shape=[jax.ShapeDtypeStruct((N2, D1), f32)] * 3 +
                  [jax.ShapeDtypeStruct((N2, 16), f32)] * 3,
    )(*xs, w1b, wa, bea)


_MESH = plsc.VectorSubcoreMesh(core_axis_name="c", subcore_axis_name="s")


@functools.partial(
    pl.kernel,
    out_type=jax.ShapeDtypeStruct((NC, N2, D1), f32),
    mesh=_MESH,
    compiler_params=pltpu.CompilerParams(use_tc_tiling_on_sc=False, needs_layout_passes=False),
    scratch_types=[
        pltpu.VMEM((40, SW), i32),
        pltpu.VMEM((40, SW), i32),
        pltpu.VMEM((SW, D1), f32),
        pltpu.VMEM((SW, D1), f32),
        pltpu.VMEM_SHARED((N2, D1), f32),
        pltpu.SemaphoreType.DMA,
        pltpu.SemaphoreType.DMA,
    ],
)
def _sc_phase1(yu, yv, yy, su, du, sv, dv, sy, dy, z80, out,
               idx_s, idx_d, buf0, buf1, acc, gsem, ssem):
    cid = lax.axis_index("c")
    sid = lax.axis_index("s")
    wid = sid * NC + cid
    pltpu.sync_copy(z80.at[pl.ds(sid * RPT, RPT)], acc.at[pl.ds(sid * RPT, RPT)])
    plsc.subcore_barrier()
    for (y, s2d, d2d) in ((yu, su, du), (yv, sv, dv), (yy, sy, dy)):
        pltpu.sync_copy(s2d.at[pl.ds(wid * 40, 40)], idx_s)
        pltpu.sync_copy(d2d.at[pl.ds(wid * 40, 40)], idx_d)

        g_issue = lambda j, b: pltpu.async_copy(y.at[idx_s.at[j]], b, gsem)
        g_wait = lambda j, b: pltpu.make_async_copy(y.at[idx_s.at[j]], b, gsem).wait()
        s_issue = lambda j, b: pltpu.async_copy(b, acc.at[idx_d.at[j]], ssem, add=True)
        s_wait = lambda j, b: pltpu.make_async_copy(b, acc.at[idx_d.at[j]], ssem).wait()

        g_issue(0, buf0)

        def pair(jj, carry):
            j0 = 2 * jj
            j1 = j0 + 1
            g_wait(j0, buf0)

            @pl.when(jj > 0)
            def _():
                s_wait(j1 - 2, buf1)

            g_issue(j1, buf1)
            s_issue(j0, buf0)
            g_wait(j1, buf1)
            s_wait(j0, buf0)

            @pl.when(jj < 19)
            def _():
                g_issue(j0 + 2, buf0)

            s_issue(j1, buf1)
            return carry

        lax.fori_loop(0, 20, pair, 0)
        s_wait(39, buf1)
    plsc.subcore_barrier()
    pltpu.sync_copy(acc.at[pl.ds(sid * RPT, RPT)],
                    out.at[cid, pl.ds(sid * RPT, RPT)])


def _tc2_body(part, b1r, b2r, w2b, wb, tu, tv, ty):
    n1 = jnp.tanh(part[0] + part[1] + b1r[...])
    n2 = jnp.tanh(jnp.dot(n1, w2b[...], preferred_element_type=f32) + b2r[...])
    for p, to in enumerate((tu, tv, ty)):
        bp = jnp.dot(n2, wb[p], preferred_element_type=f32)
        to[...] = jnp.concatenate([n2, bp], axis=1)


def _tc2(part1, b1r, b2r, w2b, wb):
    full = lambda s: pl.BlockSpec(s, lambda i: (0,) * len(s))
    return pl.pallas_call(
        _tc2_body,
        grid=(GRID,),
        in_specs=[pl.BlockSpec((NC, BN, D1), lambda i: (0, i, 0)),
                  full((1, D1)), full((1, D1)), full((D1, D1)),
                  full((3, D1, 16))],
        out_specs=[pl.BlockSpec((BN, DT), lambda i: (i, 0))] * 3,
        out_shape=[jax.ShapeDtypeStruct((N2, DT), f32)] * 3,
    )(part1, b1r, b2r, w2b, wb)


@functools.partial(
    pl.kernel,
    out_type=jax.ShapeDtypeStruct((3, NC, N2, DT), f32),
    mesh=_MESH,
    compiler_params=pltpu.CompilerParams(use_tc_tiling_on_sc=False, needs_layout_passes=False),
    scratch_types=[
        pltpu.VMEM((40, SW), i32),
        pltpu.VMEM((40, SW), i32),
        pltpu.VMEM((SW, DT), f32),
        pltpu.VMEM((SW, DT), f32),
        pltpu.VMEM((SW, 16), f32),
        pltpu.VMEM((SW, 16), f32),
        pltpu.VMEM_SHARED((N2, DT), f32),
        pltpu.SemaphoreType.DMA,
        pltpu.SemaphoreType.DMA,
        pltpu.SemaphoreType.DMA,
    ],
)
def _sc_phase2(tu, tv, ty, au, av, ay, su, du, sv, dv, sy, dy, z96, out,
               idx_s, idx_d, nbuf0, nbuf1, abuf0, abuf1, acc, asem, nsem, ssem):
    cid = lax.axis_index("c")
    sid = lax.axis_index("s")
    wid = sid * NC + cid
    lane_lt5 = lax.iota(i32, 16) < C
    cntvec = jnp.where(lax.iota(i32, 16) == (85 - D1), 1.0, 0.0).astype(f32)
    for p, (tab, atab, s2d, d2d) in enumerate(
            ((tu, au, su, du), (tv, av, sv, dv), (ty, ay, sy, dy))):
        pltpu.sync_copy(z96.at[pl.ds(sid * RPT, RPT)],
                        acc.at[pl.ds(sid * RPT, RPT)])
        plsc.subcore_barrier()
        pltpu.sync_copy(s2d.at[pl.ds(wid * 40, 40)], idx_s)
        pltpu.sync_copy(d2d.at[pl.ds(wid * 40, 40)], idx_d)

        def g_issue(j, ab, nb):
            pltpu.async_copy(atab.at[idx_s.at[j]], ab, asem)
            pltpu.async_copy(tab.at[idx_d.at[j]], nb, nsem)

        def g_wait(j, ab, nb):
            pltpu.make_async_copy(atab.at[idx_s.at[j]], ab, asem).wait()
            pltpu.make_async_copy(tab.at[idx_d.at[j]], nb, nsem).wait()

        s_issue = lambda j, nb: pltpu.async_copy(nb, acc.at[idx_s.at[j]], ssem, add=True)
        s_wait = lambda j, nb: pltpu.make_async_copy(nb, acc.at[idx_s.at[j]], ssem).wait()

        def compute(ab, nb):
            def edge(e, c2):
                a = ab[e, :]
                b = nb[e, pl.ds(D1, 16)]
                l = jnp.where(lane_lt5, a + b, -1e30)
                ex = jnp.exp(l - jnp.max(l))
                sc = ex / jnp.sum(ex)
                for c in range(C):
                    v = nb[e, pl.ds(c * NF, NF)]
                    nb[e, pl.ds(c * NF, NF)] = v * sc[c]
                nb[e, pl.ds(D1, 16)] = cntvec
                return c2

            lax.fori_loop(0, SW, edge, 0)

        g_issue(0, abuf0, nbuf0)

        def pair(jj, carry):
            j0 = 2 * jj
            j1 = j0 + 1
            g_wait(j0, abuf0, nbuf0)

            @pl.when(jj > 0)
            def _():
                s_wait(j1 - 2, nbuf1)

            g_issue(j1, abuf1, nbuf1)
            compute(abuf0, nbuf0)
            s_issue(j0, nbuf0)
            g_wait(j1, abuf1, nbuf1)
            compute(abuf1, nbuf1)
            s_wait(j0, nbuf0)

            @pl.when(jj < 19)
            def _():
                g_issue(j0 + 2, abuf0, nbuf0)

            s_issue(j1, nbuf1)
            return carry

        lax.fori_loop(0, 20, pair, 0)
        s_wait(39, nbuf1)
        plsc.subcore_barrier()
        pltpu.sync_copy(acc.at[pl.ds(sid * RPT, RPT)],
                        out.at[p, cid, pl.ds(sid * RPT, RPT)])
        plsc.subcore_barrier()


def _tc3_body(xu, xv, xy, part, wn1b, bn1r, wn2b, bn2r, ou, ov, oy):
    for p, (x, o) in enumerate(((xu, ou), (xv, ov), (xy, oy))):
        s = part[p, 0, :, :D1] + part[p, 1, :, :D1]
        cnt = part[p, 0, :, 85:86] + part[p, 1, :, 85:86]
        aggr = s / jnp.maximum(cnt, 1.0)
        h = jnp.concatenate([x[...], aggr], axis=1)
        h = jnp.tanh(jnp.dot(h, wn1b[p], preferred_element_type=f32) + bn1r[p])
        h = jnp.tanh(jnp.dot(h, wn2b[p], preferred_element_type=f32) + bn2r[p])
        o[...] = h


def _tc3(xs, part2, wn1b, bn1r, wn2b, bn2r):
    full = lambda s: pl.BlockSpec(s, lambda i: (0,) * len(s))
    return pl.pallas_call(
        _tc3_body,
        grid=(GRID,),
        in_specs=[pl.BlockSpec((BN, DX), lambda i: (i, 0))] * 3 + [
            pl.BlockSpec((3, NC, BN, DT), lambda i: (0, 0, i, 0)),
            full((3, DX + D1, DX)), full((3, 1, DX)),
            full((3, DX, DX)), full((3, 1, DX))],
        out_specs=[pl.BlockSpec((BN, DX), lambda i: (i, 0))] * 3,
        out_shape=[jax.ShapeDtypeStruct((N2, DX), f32)] * 3,
    )(*xs, part2, wn1b, bn1r, wn2b, bn2r)


def kernel(x_u, x_v, x_y, edge_index_u, edge_index_v, edge_index_y, nexus,
           W1, b1, W2, b2, We, be, Wn1, bn1, Wn2, bn2):
    E = edge_index_u.shape[1]
    E2 = ((E + NW * SW - 1) // (NW * SW)) * (NW * SW)

    xs = [jnp.pad(x.reshape(N, DX), ((0, N2 - N), (0, 0)))
          for x in (x_u, x_v, x_y)]
    eidx = []
    for ei in (edge_index_u, edge_index_v, edge_index_y):
        s = jnp.pad(ei[0], (0, E2 - E), constant_values=N).reshape(-1, SW)
        d = jnp.pad(ei[1], (0, E2 - E), constant_values=N).reshape(-1, SW)
        eidx += [s, d]
    z80 = jnp.zeros((N2, D1), f32)
    z96 = jnp.zeros((N2, DT), f32)

    w1b = jnp.stack([_blkdiag(W1[:, p * PF:(p + 1) * PF, :]) for p in range(3)])
    wa = jnp.stack([
        jnp.pad(_blkdiag(We[p, :, :PF, :]), ((0, 0), (0, 16 - C)))
        for p in range(3)])
    bea = jnp.pad(be[:, :, 0], ((0, 0), (0, 16 - C)))
    b1r = b1.reshape(1, D1)
    b2r = b2.reshape(1, D1)
    w2b = _blkdiag(W2)
    wb = jnp.stack([
        jnp.pad(_blkdiag(We[p, :, PF:, :]), ((0, 0), (0, 16 - C)))
        for p in range(3)])
    perm = np.empty((DX + D1,), np.int32)
    for c in range(C):
        perm[c * PF:(c + 1) * PF] = c * (PF + NF) + np.arange(PF)
        perm[DX + c * NF:DX + (c + 1) * NF] = c * (PF + NF) + PF + np.arange(NF)
    wn1b = jnp.stack([jnp.take(_blkdiag(Wn1[p]), perm, axis=0)
                      for p in range(3)])
    bn1r = bn1.reshape(3, 1, DX)
    wn2b = jnp.stack([_blkdiag(Wn2[p]) for p in range(3)])
    bn2r = bn2.reshape(3, 1, DX)

    yu, yv, yy, au, av, ay = _tc1(xs, w1b, wa, bea)
    part1 = _sc_phase1(yu, yv, yy, *eidx, z80)
    ntabs = _tc2(part1, b1r, b2r, w2b, wb)
    part2 = _sc_phase2(*ntabs, au, av, ay, *eidx, z96)
    outs = _tc3(xs, part2, wn1b, bn1r, wn2b, bn2r)
    return tuple(o[:N].reshape(N, C, PF) for o in outs)


# speedup vs baseline: 57.5159x; 1.0107x over previous
import functools
import numpy as np
import jax
import jax.numpy as jnp
from jax import lax
from jax.experimental import pallas as pl
from jax.experimental.pallas import tpu as pltpu
from jax.experimental.pallas import tpu_sc as plsc

N = 10000
C = 5
PF = 64
NF = 16
D1 = C * NF
DT = 96
DX = C * PF
NC = 2
NS = 16
NW = NC * NS
SW = 128
N2 = 10240
RPT = N2 // NS
BN = 400
NGRID = N // BN
BN2 = 512
GRID = N2 // BN2

f32 = jnp.float32
i32 = jnp.int32


def _blkdiag(w):
    return jax.scipy.linalg.block_diag(*[w[c] for c in range(w.shape[0])])


def _tc1_body(xu, xv, xy, w1p, wea, bea, yu, yv, yy, au, av, ay):
    for p, (x, yo, ao) in enumerate(((xu, yu, au), (xv, yv, av), (xy, yy, ay))):
        ycs = []
        acs = []
        for c in range(C):
            xc = x[:, c, :]
            ycs.append(jnp.dot(xc, w1p[p, c], preferred_element_type=f32))
            acs.append(jnp.sum(xc * wea[p, c], axis=1, keepdims=True)
                       + bea[p, c])
        yo[...] = jnp.concatenate(ycs, axis=1)
        ao[...] = jnp.concatenate(acs + [jnp.zeros((BN, 16 - C), f32)], axis=1)


def _tc1(xs, w1p, wea, bea):
    full = lambda s: pl.BlockSpec(s, lambda i: (0,) * len(s))
    return pl.pallas_call(
        _tc1_body,
        grid=(NGRID,),
        in_specs=[pl.BlockSpec((BN, C, PF), lambda i: (i, 0, 0))] * 3 + [
            full((3, C, PF, NF)), full((3, C, PF)), full((3, C))],
        out_specs=[pl.BlockSpec((BN, D1), lambda i: (i, 0))] * 3 +
                  [pl.BlockSpec((BN, 16), lambda i: (i, 0))] * 3,
        out_shape=[jax.ShapeDtypeStruct((N2, D1), f32)] * 3 +
                  [jax.ShapeDtypeStruct((N2, 16), f32)] * 3,
    )(*xs, w1p, wea, bea)


_MESH = plsc.VectorSubcoreMesh(core_axis_name="c", subcore_axis_name="s")


@functools.partial(
    pl.kernel,
    out_type=jax.ShapeDtypeStruct((NC, N2, D1), f32),
    mesh=_MESH,
    compiler_params=pltpu.CompilerParams(use_tc_tiling_on_sc=False, needs_layout_passes=False),
    scratch_types=[
        pltpu.VMEM((40, SW), i32),
        pltpu.VMEM((40, SW), i32),
        pltpu.VMEM((SW, D1), f32),
        pltpu.VMEM((SW, D1), f32),
        pltpu.VMEM_SHARED((N2, D1), f32),
        pltpu.SemaphoreType.DMA,
        pltpu.SemaphoreType.DMA,
    ],
)
def _sc_phase1(yu, yv, yy, su, du, sv, dv, sy, dy, z80, out,
               idx_s, idx_d, buf0, buf1, acc, gsem, ssem):
    cid = lax.axis_index("c")
    sid = lax.axis_index("s")
    wid = sid * NC + cid
    pltpu.sync_copy(z80.at[pl.ds(sid * RPT, RPT)], acc.at[pl.ds(sid * RPT, RPT)])
    plsc.subcore_barrier()
    for (y, s2d, d2d) in ((yu, su, du), (yv, sv, dv), (yy, sy, dy)):
        pltpu.sync_copy(s2d.at[pl.ds(wid * 40, 40)], idx_s)
        pltpu.sync_copy(d2d.at[pl.ds(wid * 40, 40)], idx_d)

        g_issue = lambda j, b: pltpu.async_copy(y.at[idx_s.at[j]], b, gsem)
        g_wait = lambda j, b: pltpu.make_async_copy(y.at[idx_s.at[j]], b, gsem).wait()
        s_issue = lambda j, b: pltpu.async_copy(b, acc.at[idx_d.at[j]], ssem, add=True)
        s_wait = lambda j, b: pltpu.make_async_copy(b, acc.at[idx_d.at[j]], ssem).wait()

        g_issue(0, buf0)

        def pair(jj, carry):
            j0 = 2 * jj
            j1 = j0 + 1
            g_wait(j0, buf0)

            @pl.when(jj > 0)
            def _():
                s_wait(j1 - 2, buf1)

            g_issue(j1, buf1)
            s_issue(j0, buf0)
            g_wait(j1, buf1)
            s_wait(j0, buf0)

            @pl.when(jj < 19)
            def _():
                g_issue(j0 + 2, buf0)

            s_issue(j1, buf1)
            return carry

        lax.fori_loop(0, 20, pair, 0)
        s_wait(39, buf1)
    plsc.subcore_barrier()
    pltpu.sync_copy(acc.at[pl.ds(sid * RPT, RPT)],
                    out.at[cid, pl.ds(sid * RPT, RPT)])


def _tc2_body(part, b1r, b2r, w2b, wb, tu, tv, ty):
    n1 = jnp.tanh(part[0] + part[1] + b1r[...])
    n2 = jnp.tanh(jnp.dot(n1, w2b[...], preferred_element_type=f32) + b2r[...])
    for p, to in enumerate((tu, tv, ty)):
        bp = jnp.dot(n2, wb[p], preferred_element_type=f32)
        to[...] = jnp.concatenate([n2, bp], axis=1)


def _tc2(part1, b1r, b2r, w2b, wb):
    full = lambda s: pl.BlockSpec(s, lambda i: (0,) * len(s))
    return pl.pallas_call(
        _tc2_body,
        grid=(GRID,),
        in_specs=[pl.BlockSpec((NC, BN2, D1), lambda i: (0, i, 0)),
                  full((1, D1)), full((1, D1)), full((D1, D1)),
                  full((3, D1, 16))],
        out_specs=[pl.BlockSpec((BN2, DT), lambda i: (i, 0))] * 3,
        out_shape=[jax.ShapeDtypeStruct((N2, DT), f32)] * 3,
    )(part1, b1r, b2r, w2b, wb)


@functools.partial(
    pl.kernel,
    out_type=jax.ShapeDtypeStruct((3, NC, N2, DT), f32),
    mesh=_MESH,
    compiler_params=pltpu.CompilerParams(use_tc_tiling_on_sc=False, needs_layout_passes=False),
    scratch_types=[
        pltpu.VMEM((40, SW), i32),
        pltpu.VMEM((40, SW), i32),
        pltpu.VMEM((SW, DT), f32),
        pltpu.VMEM((SW, DT), f32),
        pltpu.VMEM((SW, 16), f32),
        pltpu.VMEM((SW, 16), f32),
        pltpu.VMEM_SHARED((N2, DT), f32),
        pltpu.SemaphoreType.DMA,
        pltpu.SemaphoreType.DMA,
        pltpu.SemaphoreType.DMA,
    ],
)
def _sc_phase2(tu, tv, ty, au, av, ay, su, du, sv, dv, sy, dy, z96, out,
               idx_s, idx_d, nbuf0, nbuf1, abuf0, abuf1, acc, asem, nsem, ssem):
    cid = lax.axis_index("c")
    sid = lax.axis_index("s")
    wid = sid * NC + cid
    lane_lt5 = lax.iota(i32, 16) < C
    cntvec = jnp.where(lax.iota(i32, 16) == (85 - D1), 1.0, 0.0).astype(f32)
    for p, (tab, atab, s2d, d2d) in enumerate(
            ((tu, au, su, du), (tv, av, sv, dv), (ty, ay, sy, dy))):
        pltpu.sync_copy(z96.at[pl.ds(sid * RPT, RPT)],
                        acc.at[pl.ds(sid * RPT, RPT)])
        plsc.subcore_barrier()
        pltpu.sync_copy(s2d.at[pl.ds(wid * 40, 40)], idx_s)
        pltpu.sync_copy(d2d.at[pl.ds(wid * 40, 40)], idx_d)

        def g_issue(j, ab, nb):
            pltpu.async_copy(atab.at[idx_s.at[j]], ab, asem)
            pltpu.async_copy(tab.at[idx_d.at[j]], nb, nsem)

        def g_wait(j, ab, nb):
            pltpu.make_async_copy(atab.at[idx_s.at[j]], ab, asem).wait()
            pltpu.make_async_copy(tab.at[idx_d.at[j]], nb, nsem).wait()

        s_issue = lambda j, nb: pltpu.async_copy(nb, acc.at[idx_s.at[j]], ssem, add=True)
        s_wait = lambda j, nb: pltpu.make_async_copy(nb, acc.at[idx_s.at[j]], ssem).wait()

        def compute(ab, nb):
            def edge(e, c2):
                a = ab[e, :]
                b = nb[e, pl.ds(D1, 16)]
                l = jnp.where(lane_lt5, a + b, -1e30)
                ex = jnp.exp(l - jnp.max(l))
                sc = ex / jnp.sum(ex)
                for c in range(C):
                    v = nb[e, pl.ds(c * NF, NF)]
                    nb[e, pl.ds(c * NF, NF)] = v * sc[c]
                nb[e, pl.ds(D1, 16)] = cntvec
                return c2

            lax.fori_loop(0, SW, edge, 0)

        g_issue(0, abuf0, nbuf0)

        def pair(jj, carry):
            j0 = 2 * jj
            j1 = j0 + 1
            g_wait(j0, abuf0, nbuf0)

            @pl.when(jj > 0)
            def _():
                s_wait(j1 - 2, nbuf1)

            g_issue(j1, abuf1, nbuf1)
            compute(abuf0, nbuf0)
            s_issue(j0, nbuf0)
            g_wait(j1, abuf1, nbuf1)
            compute(abuf1, nbuf1)
            s_wait(j0, nbuf0)

            @pl.when(jj < 19)
            def _():
                g_issue(j0 + 2, abuf0, nbuf0)

            s_issue(j1, nbuf1)
            return carry

        lax.fori_loop(0, 20, pair, 0)
        s_wait(39, nbuf1)
        plsc.subcore_barrier()
        pltpu.sync_copy(acc.at[pl.ds(sid * RPT, RPT)],
                        out.at[p, cid, pl.ds(sid * RPT, RPT)])
        plsc.subcore_barrier()


def _tc3_body(xu, xv, xy, part, wn1, bn1, wn2, bn2, ou, ov, oy):
    for p, (x, o) in enumerate(((xu, ou), (xv, ov), (xy, oy))):
        cnt = part[p, 0, :, 85:86] + part[p, 1, :, 85:86]
        inv = 1.0 / jnp.maximum(cnt, 1.0)
        h2s = []
        for c in range(C):
            s = (part[p, 0, :, c * NF:(c + 1) * NF] +
                 part[p, 1, :, c * NF:(c + 1) * NF])
            aggr = s * inv
            h1 = jnp.tanh(
                jnp.dot(x[:, c, :], wn1[p, c, :PF, :],
                        preferred_element_type=f32) +
                jnp.dot(aggr, wn1[p, c, PF:, :], preferred_element_type=f32) +
                bn1[p, c])
            h2 = jnp.tanh(
                jnp.dot(h1, wn2[p, c], preferred_element_type=f32) + bn2[p, c])
            h2s.append(h2[:, None, :])
        o[...] = jnp.concatenate(h2s, axis=1)


def _tc3(xs, part2, wn1, bn1, wn2, bn2):
    full = lambda s: pl.BlockSpec(s, lambda i: (0,) * len(s))
    return pl.pallas_call(
        _tc3_body,
        grid=(NGRID,),
        in_specs=[pl.BlockSpec((BN, C, PF), lambda i: (i, 0, 0))] * 3 + [
            pl.BlockSpec((3, NC, BN, DT), lambda i: (0, 0, i, 0)),
            full((3, C, PF + NF, PF)), full((3, C, PF)),
            full((3, C, PF, PF)), full((3, C, PF))],
        out_specs=[pl.BlockSpec((BN, C, PF), lambda i: (i, 0, 0))] * 3,
        out_shape=[jax.ShapeDtypeStruct((N, C, PF), f32)] * 3,
    )(*xs, part2, wn1, bn1, wn2, bn2)


def kernel(x_u, x_v, x_y, edge_index_u, edge_index_v, edge_index_y, nexus,
           W1, b1, W2, b2, We, be, Wn1, bn1, Wn2, bn2):
    E = edge_index_u.shape[1]
    E2 = ((E + NW * SW - 1) // (NW * SW)) * (NW * SW)

    xs = [x_u, x_v, x_y]
    eidx = []
    for ei in (edge_index_u, edge_index_v, edge_index_y):
        s = jnp.pad(ei[0], (0, E2 - E), constant_values=N).reshape(-1, SW)
        d = jnp.pad(ei[1], (0, E2 - E), constant_values=N).reshape(-1, SW)
        eidx += [s, d]
    z80 = jnp.zeros((N2, D1), f32)
    z96 = jnp.zeros((N2, DT), f32)

    w1p = jnp.stack([W1[:, p * PF:(p + 1) * PF, :] for p in range(3)])
    wea = We[:, :, :PF, 0]
    bea = be[:, :, 0]
    b1r = b1.reshape(1, D1)
    b2r = b2.reshape(1, D1)
    w2b = _blkdiag(W2)
    wb = jnp.stack([
        jnp.pad(_blkdiag(We[p, :, PF:, :]), ((0, 0), (0, 16 - C)))
        for p in range(3)])

    yu, yv, yy, au, av, ay = _tc1(xs, w1p, wea, bea)
    part1 = _sc_phase1(yu, yv, yy, *eidx, z80)
    ntabs = _tc2(part1, b1r, b2r, w2b, wb)
    part2 = _sc_phase2(*ntabs, au, av, ay, *eidx, z96)
    return _tc3(xs, part2, Wn1, bn1, Wn2, bn2)
